# baseline (speedup 1.0000x reference)
import jax
import jax.numpy as jnp
from jax import lax
from jax.experimental import pallas as pl
from jax.experimental.pallas import tpu as pltpu
from jax.experimental.pallas import tpu_sc as plsc

N = 10000
D = 64
K = 32
NC, NS = 2, 16
NPAD = 10240
CN = 32
NB0 = 512
NB1 = 128
NCH0 = NB0 // CN
NCH1 = NB1 // CN
ROWS = CN * K
CORE0_NODES = NS * NB0

_SQRT2 = 1.4142135623730951
_F32 = jnp.float32
_BF16 = jnp.bfloat16


def _gelu(v):
    return v * 0.5 * (1.0 + lax.erf(v / _SQRT2))


def _bn(m, g, b):
    mu = jnp.mean(m, axis=0, keepdims=True)
    var = jnp.mean((m - mu) ** 2, axis=0, keepdims=True)
    return (m - mu) * lax.rsqrt(var + 1e-5) * g + b


_DIMS = (((1,), (1,)), ((), ()))


def _mm_body(xt_ref, w_ref, y_ref, z_ref):
    xt = xt_ref[...]
    w = w_ref[...]
    a = w[:, :D]
    bz = w[:, D:] - a
    y_ref[...] = lax.dot_general(
        xt, a, _DIMS, preferred_element_type=_F32).astype(_BF16)
    z_ref[...] = lax.dot_general(xt, bz, _DIMS, preferred_element_type=_F32)


_mm = pl.pallas_call(
    _mm_body,
    out_shape=(jax.ShapeDtypeStruct((N, D), _BF16),
               jax.ShapeDtypeStruct((N, D), _F32)),
)


def _bnmm_body(mx_ref, z_ref, w_ref, g_ref, b_ref, y2_ref, z2_ref):
    m = mx_ref[...].astype(_F32) + z_ref[...]
    h = _gelu(_bn(m, g_ref[...], b_ref[...]))
    w = w_ref[...]
    a = w[:, :D]
    bz = w[:, D:] - a
    y2_ref[...] = lax.dot_general(
        h, a, _DIMS, preferred_element_type=_F32).astype(_BF16)
    z2_ref[...] = lax.dot_general(h, bz, _DIMS, preferred_element_type=_F32)


_bnmm = pl.pallas_call(
    _bnmm_body,
    out_shape=(jax.ShapeDtypeStruct((N, D), _BF16),
               jax.ShapeDtypeStruct((N, D), _F32)),
)


def _final_body(mx_ref, z_ref, xt_ref, g_ref, b_ref, o_ref):
    s = xt_ref[...] + mx_ref[...].astype(_F32) + z_ref[...]
    o_ref[...] = _gelu(_bn(s, g_ref[...], b_ref[...]))


_final = pl.pallas_call(
    _final_body,
    out_shape=jax.ShapeDtypeStruct((N, D), _F32),
)


def _sc_gathermax_body(y_hbm, nbr_hbm, out_hbm, idx_v,
                       g0, g1, ob0, ob1, sg0, sg1, so0, so1):
    cid = lax.axis_index("c")
    sid = lax.axis_index("s")
    node_base = jnp.where(cid == 0, sid * NB0, CORE0_NODES + sid * NB1)
    npairs = jnp.where(cid == 0, NCH0 // 2, NCH1 // 2)
    idx_base = node_base * K

    gs = (g0, g1)
    obs = (ob0, ob1)
    sgs = (sg0, sg1)
    sos = (so0, so1)

    @pl.when(cid == 0)
    def _copy_idx0():
        pltpu.sync_copy(nbr_hbm.at[pl.ds(idx_base, NB0 * K)], idx_v)

    @pl.when(cid == 1)
    def _copy_idx1():
        pltpu.sync_copy(nbr_hbm.at[pl.ds(idx_base, NB1 * K)],
                        idx_v.at[pl.ds(0, NB1 * K)])

    def issue(c, b):
        pltpu.async_copy(y_hbm.at[idx_v.at[pl.ds(c * ROWS, ROWS)]],
                         gs[b], sgs[b])

    issue(0, 0)
    issue(1, 1)

    def pair(c2, carry):
        for b in range(2):
            c = 2 * c2 + b
            gbuf = gs[b]
            obuf = obs[b]
            pltpu.make_async_copy(y_hbm.at[pl.ds(0, ROWS)], gbuf,
                                  sgs[b]).wait()

            @pl.when(c2 > 0)
            def _drain_store():
                pltpu.make_async_copy(obuf, out_hbm.at[pl.ds(0, CN)],
                                      sos[b]).wait()

            def node(i, carry2):
                base = i * K
                for q in range(D // 32):
                    m = gbuf[base, pl.ds(q * 32, 32)]
                    for r in range(1, K):
                        m = jnp.maximum(m, gbuf[base + r, pl.ds(q * 32, 32)])
                    obuf[i, pl.ds(q * 32, 32)] = m
                return carry2

            lax.fori_loop(0, CN, node, 0)
            row0 = node_base + c * CN
            pltpu.async_copy(obuf, out_hbm.at[pl.ds(row0, CN)], sos[b])

            nxt = c + 2

            @pl.when(nxt < 2 * npairs)
            def _issue_next():
                issue(nxt, b)
        return carry

    lax.fori_loop(0, npairs, pair, 0)
    for b in range(2):
        pltpu.make_async_copy(obs[b], out_hbm.at[pl.ds(0, CN)],
                              sos[b]).wait()


_gathermax = pl.kernel(
    _sc_gathermax_body,
    out_type=jax.ShapeDtypeStruct((NPAD, D), _BF16),
    mesh=plsc.VectorSubcoreMesh(core_axis_name="c", subcore_axis_name="s",
                                num_cores=NC, num_subcores=NS),
    scratch_types=[
        pltpu.VMEM((NB0 * K,), jnp.int32),
        pltpu.VMEM((ROWS, D), _BF16),
        pltpu.VMEM((ROWS, D), _BF16),
        pltpu.VMEM((CN, D), _BF16),
        pltpu.VMEM((CN, D), _BF16),
        pltpu.SemaphoreType.DMA,
        pltpu.SemaphoreType.DMA,
        pltpu.SemaphoreType.DMA,
        pltpu.SemaphoreType.DMA,
    ],
    compiler_params=pltpu.CompilerParams(use_tc_tiling_on_sc=False),
)


def kernel(x, neighbor_ind, W1, W2, gamma1, beta1, gamma2, beta2):
    xt = x[0].T
    nbr = neighbor_ind[0].astype(jnp.int32)
    nbr_flat = jnp.pad(nbr, ((0, NPAD - N), (0, 0))).reshape(-1)
    g1 = gamma1.reshape(1, D)
    b1 = beta1.reshape(1, D)
    g2 = gamma2.reshape(1, D)
    b2 = beta2.reshape(1, D)

    y1, z1 = _mm(xt, W1)
    mx1 = _gathermax(y1, nbr_flat)[:N]
    y2, z2 = _bnmm(mx1, z1, W2, g1, b1)
    mx2 = _gathermax(y2, nbr_flat)[:N]
    out_t = _final(mx2, z2, xt, g2, b2)
    return out_t.T[None]


# speedup vs baseline: 1.7854x; 1.7854x over previous
import jax
import jax.numpy as jnp
from jax import lax
from jax.experimental import pallas as pl
from jax.experimental.pallas import tpu as pltpu
from jax.experimental.pallas import tpu_sc as plsc

N = 10000
D = 64
K = 32
NC, NS = 2, 16
NPAD = 10240
CN = 32
NB0 = 576
NB1 = 64
NCH0 = NB0 // CN
NCH1 = NB1 // CN
ROWS = CN * K
CORE0_NODES = NS * NB0

_SQRT2 = 1.4142135623730951
_F32 = jnp.float32
_BF16 = jnp.bfloat16


def _gelu(v):
    return v * 0.5 * (1.0 + lax.erf(v / _SQRT2))


def _bn(m, g, b):
    mu = jnp.mean(m, axis=0, keepdims=True)
    var = jnp.mean((m - mu) ** 2, axis=0, keepdims=True)
    return (m - mu) * lax.rsqrt(var + 1e-5) * g + b


_DIMS = (((1,), (1,)), ((), ()))


def _mm_body(xt_ref, w_ref, y_ref, z_ref):
    xt = xt_ref[...]
    w = w_ref[...]
    a = w[:, :D]
    bz = w[:, D:] - a
    y_ref[...] = lax.dot_general(
        xt, a, _DIMS, preferred_element_type=_F32).astype(_BF16)
    z_ref[...] = lax.dot_general(xt, bz, _DIMS, preferred_element_type=_F32)


_mm = pl.pallas_call(
    _mm_body,
    out_shape=(jax.ShapeDtypeStruct((N, D), _BF16),
               jax.ShapeDtypeStruct((N, D), _F32)),
)


def _bnmm_body(mx_ref, z_ref, w_ref, g_ref, b_ref, y2_ref, z2_ref):
    m = mx_ref[...].astype(_F32) + z_ref[...]
    h = _gelu(_bn(m, g_ref[...], b_ref[...]))
    w = w_ref[...]
    a = w[:, :D]
    bz = w[:, D:] - a
    y2_ref[...] = lax.dot_general(
        h, a, _DIMS, preferred_element_type=_F32).astype(_BF16)
    z2_ref[...] = lax.dot_general(h, bz, _DIMS, preferred_element_type=_F32)


_bnmm = pl.pallas_call(
    _bnmm_body,
    out_shape=(jax.ShapeDtypeStruct((N, D), _BF16),
               jax.ShapeDtypeStruct((N, D), _F32)),
)


def _final_body(mx_ref, z_ref, xt_ref, g_ref, b_ref, o_ref):
    s = xt_ref[...] + mx_ref[...].astype(_F32) + z_ref[...]
    o_ref[...] = _gelu(_bn(s, g_ref[...], b_ref[...]))


_final = pl.pallas_call(
    _final_body,
    out_shape=jax.ShapeDtypeStruct((N, D), _F32),
)


def _sc_gathermax_body(y_hbm, nbr_hbm, out_hbm, idx_v,
                       tbl, g0, g1, ob0, ob1, sg0, sg1, so0, so1):
    cid = lax.axis_index("c")
    sid = lax.axis_index("s")
    node_base = jnp.where(cid == 0, sid * NB0, CORE0_NODES + sid * NB1)
    npairs = jnp.where(cid == 0, NCH0 // 2, NCH1 // 2)
    idx_base = node_base * K

    gs = (g0, g1)
    obs = (ob0, ob1)
    sgs = (sg0, sg1)
    sos = (so0, so1)

    @pl.when(cid == 0)
    def _copy_idx0():
        pltpu.sync_copy(nbr_hbm.at[pl.ds(idx_base, NB0 * K)], idx_v)

    @pl.when(cid == 1)
    def _copy_idx1():
        pltpu.sync_copy(nbr_hbm.at[pl.ds(idx_base, NB1 * K)],
                        idx_v.at[pl.ds(0, NB1 * K)])

    pltpu.sync_copy(y_hbm.at[pl.ds(sid * (N // NS), N // NS)],
                    tbl.at[pl.ds(sid * (N // NS), N // NS)])
    plsc.subcore_barrier()

    def issue(c, b):
        pltpu.async_copy(tbl.at[idx_v.at[pl.ds(c * ROWS, ROWS)]],
                         gs[b], sgs[b])

    issue(0, 0)
    issue(1, 1)

    def pair(c2, carry):
        for b in range(2):
            c = 2 * c2 + b
            gbuf = gs[b]
            obuf = obs[b]
            pltpu.make_async_copy(y_hbm.at[pl.ds(0, ROWS)], gbuf,
                                  sgs[b]).wait()

            @pl.when(c2 > 0)
            def _drain_store():
                pltpu.make_async_copy(obuf, out_hbm.at[pl.ds(0, CN)],
                                      sos[b]).wait()

            def node(i, carry2):
                base = i * K
                for q in range(D // 32):
                    m = gbuf[base, pl.ds(q * 32, 32)]
                    for r in range(1, K):
                        m = jnp.maximum(m, gbuf[base + r, pl.ds(q * 32, 32)])
                    obuf[i, pl.ds(q * 32, 32)] = m
                return carry2

            lax.fori_loop(0, CN, node, 0)
            row0 = node_base + c * CN
            pltpu.async_copy(obuf, out_hbm.at[pl.ds(row0, CN)], sos[b])

            nxt = c + 2

            @pl.when(nxt < 2 * npairs)
            def _issue_next():
                issue(nxt, b)
        return carry

    lax.fori_loop(0, npairs, pair, 0)
    for b in range(2):
        pltpu.make_async_copy(obs[b], out_hbm.at[pl.ds(0, CN)],
                              sos[b]).wait()


_gathermax = pl.kernel(
    _sc_gathermax_body,
    out_type=jax.ShapeDtypeStruct((NPAD, D), _BF16),
    mesh=plsc.VectorSubcoreMesh(core_axis_name="c", subcore_axis_name="s",
                                num_cores=NC, num_subcores=NS),
    scratch_types=[
        pltpu.VMEM((NB0 * K,), jnp.int32),
        pltpu.VMEM_SHARED((N, D), _BF16),
        pltpu.VMEM((ROWS, D), _BF16),
        pltpu.VMEM((ROWS, D), _BF16),
        pltpu.VMEM((CN, D), _BF16),
        pltpu.VMEM((CN, D), _BF16),
        pltpu.SemaphoreType.DMA,
        pltpu.SemaphoreType.DMA,
        pltpu.SemaphoreType.DMA,
        pltpu.SemaphoreType.DMA,
    ],
    compiler_params=pltpu.CompilerParams(use_tc_tiling_on_sc=False),
)


def kernel(x, neighbor_ind, W1, W2, gamma1, beta1, gamma2, beta2):
    xt = x[0].T
    nbr = neighbor_ind[0].astype(jnp.int32)
    nbr_flat = jnp.pad(nbr, ((0, NPAD - N), (0, 0))).reshape(-1)
    g1 = gamma1.reshape(1, D)
    b1 = beta1.reshape(1, D)
    g2 = gamma2.reshape(1, D)
    b2 = beta2.reshape(1, D)

    y1, z1 = _mm(xt, W1)
    mx1 = _gathermax(y1, nbr_flat)[:N]
    y2, z2 = _bnmm(mx1, z1, W2, g1, b1)
    mx2 = _gathermax(y2, nbr_flat)[:N]
    out_t = _final(mx2, z2, xt, g2, b2)
    return out_t.T[None]


# speedup vs baseline: 2.1448x; 1.2013x over previous
import jax
import jax.numpy as jnp
from jax import lax
from jax.experimental import pallas as pl
from jax.experimental.pallas import tpu as pltpu
from jax.experimental.pallas import tpu_sc as plsc

N = 10000
D = 64
K = 32
NC, NS = 2, 16
NPAD = 10240
CN = 32
NB0 = 320
NB1 = 320
NCH0 = NB0 // CN
NCH1 = NB1 // CN
ROWS = CN * K
CORE0_NODES = NS * NB0

_SQRT2 = 1.4142135623730951
_F32 = jnp.float32
_BF16 = jnp.bfloat16


def _gelu(v):
    return v * 0.5 * (1.0 + lax.erf(v / _SQRT2))


def _bn(m, g, b):
    mu = jnp.mean(m, axis=0, keepdims=True)
    var = jnp.mean((m - mu) ** 2, axis=0, keepdims=True)
    return (m - mu) * lax.rsqrt(var + 1e-5) * g + b


_DIMS = (((1,), (1,)), ((), ()))


def _mm_body(xt_ref, w_ref, y_ref, z_ref):
    xt = xt_ref[...]
    w = w_ref[...]
    a = w[:, :D]
    bz = w[:, D:] - a
    y_ref[...] = lax.dot_general(
        xt, a, _DIMS, preferred_element_type=_F32).astype(_BF16)
    z_ref[...] = lax.dot_general(xt, bz, _DIMS, preferred_element_type=_F32)


_mm = pl.pallas_call(
    _mm_body,
    out_shape=(jax.ShapeDtypeStruct((N, D), _BF16),
               jax.ShapeDtypeStruct((N, D), _F32)),
)


def _bnmm_body(mx_ref, z_ref, w_ref, g_ref, b_ref, y2_ref, z2_ref):
    m = mx_ref[...].astype(_F32) + z_ref[...]
    h = _gelu(_bn(m, g_ref[...], b_ref[...]))
    w = w_ref[...]
    a = w[:, :D]
    bz = w[:, D:] - a
    y2_ref[...] = lax.dot_general(
        h, a, _DIMS, preferred_element_type=_F32).astype(_BF16)
    z2_ref[...] = lax.dot_general(h, bz, _DIMS, preferred_element_type=_F32)


_bnmm = pl.pallas_call(
    _bnmm_body,
    out_shape=(jax.ShapeDtypeStruct((N, D), _BF16),
               jax.ShapeDtypeStruct((N, D), _F32)),
)


def _final_body(mx_ref, z_ref, xt_ref, g_ref, b_ref, o_ref):
    s = xt_ref[...] + mx_ref[...].astype(_F32) + z_ref[...]
    o_ref[...] = _gelu(_bn(s, g_ref[...], b_ref[...]))


_final = pl.pallas_call(
    _final_body,
    out_shape=jax.ShapeDtypeStruct((N, D), _F32),
)


def _sc_gathermax_body(y_hbm, nbr_hbm, out_hbm, idx_v,
                       tbl, g0, g1, ob0, ob1, sg0, sg1, so0, so1):
    cid = lax.axis_index("c")
    sid = lax.axis_index("s")
    node_base = jnp.where(cid == 0, sid * NB0, CORE0_NODES + sid * NB1)
    npairs = jnp.where(cid == 0, NCH0 // 2, NCH1 // 2)
    idx_base = node_base * K

    gs = (g0, g1)
    obs = (ob0, ob1)
    sgs = (sg0, sg1)
    sos = (so0, so1)

    @pl.when(cid == 0)
    def _copy_idx0():
        pltpu.sync_copy(nbr_hbm.at[pl.ds(idx_base, NB0 * K)], idx_v)

    @pl.when(cid == 1)
    def _copy_idx1():
        pltpu.sync_copy(nbr_hbm.at[pl.ds(idx_base, NB1 * K)],
                        idx_v.at[pl.ds(0, NB1 * K)])

    pltpu.sync_copy(y_hbm.at[pl.ds(sid * (N // NS), N // NS)],
                    tbl.at[pl.ds(sid * (N // NS), N // NS)])
    plsc.subcore_barrier()

    def issue(c, b):
        pltpu.async_copy(tbl.at[idx_v.at[pl.ds(c * ROWS, ROWS)]],
                         gs[b], sgs[b])

    issue(0, 0)
    issue(1, 1)

    def pair(c2, carry):
        for b in range(2):
            c = 2 * c2 + b
            gbuf = gs[b]
            obuf = obs[b]
            pltpu.make_async_copy(y_hbm.at[pl.ds(0, ROWS)], gbuf,
                                  sgs[b]).wait()

            @pl.when(c2 > 0)
            def _drain_store():
                pltpu.make_async_copy(obuf, out_hbm.at[pl.ds(0, CN)],
                                      sos[b]).wait()

            def node(i, carry2):
                base = i * K
                for q in range(D // 32):
                    m = gbuf[base, pl.ds(q * 32, 32)]
                    for r in range(1, K):
                        m = jnp.maximum(m, gbuf[base + r, pl.ds(q * 32, 32)])
                    obuf[i, pl.ds(q * 32, 32)] = m
                return carry2

            lax.fori_loop(0, CN, node, 0)
            row0 = node_base + c * CN
            pltpu.async_copy(obuf, out_hbm.at[pl.ds(row0, CN)], sos[b])

            nxt = c + 2

            @pl.when(nxt < 2 * npairs)
            def _issue_next():
                issue(nxt, b)
        return carry

    lax.fori_loop(0, npairs, pair, 0)
    for b in range(2):
        pltpu.make_async_copy(obs[b], out_hbm.at[pl.ds(0, CN)],
                              sos[b]).wait()


_gathermax = pl.kernel(
    _sc_gathermax_body,
    out_type=jax.ShapeDtypeStruct((NPAD, D), _BF16),
    mesh=plsc.VectorSubcoreMesh(core_axis_name="c", subcore_axis_name="s",
                                num_cores=NC, num_subcores=NS),
    scratch_types=[
        pltpu.VMEM((NB0 * K,), jnp.int32),
        pltpu.VMEM_SHARED((N, D), _BF16),
        pltpu.VMEM((ROWS, D), _BF16),
        pltpu.VMEM((ROWS, D), _BF16),
        pltpu.VMEM((CN, D), _BF16),
        pltpu.VMEM((CN, D), _BF16),
        pltpu.SemaphoreType.DMA,
        pltpu.SemaphoreType.DMA,
        pltpu.SemaphoreType.DMA,
        pltpu.SemaphoreType.DMA,
    ],
    compiler_params=pltpu.CompilerParams(use_tc_tiling_on_sc=False),
)


def kernel(x, neighbor_ind, W1, W2, gamma1, beta1, gamma2, beta2):
    xt = x[0].T
    nbr = neighbor_ind[0].astype(jnp.int32)
    nbr_flat = jnp.pad(nbr, ((0, NPAD - N), (0, 0))).reshape(-1)
    g1 = gamma1.reshape(1, D)
    b1 = beta1.reshape(1, D)
    g2 = gamma2.reshape(1, D)
    b2 = beta2.reshape(1, D)

    y1, z1 = _mm(xt, W1)
    mx1 = _gathermax(y1, nbr_flat)[:N]
    y2, z2 = _bnmm(mx1, z1, W2, g1, b1)
    mx2 = _gathermax(y2, nbr_flat)[:N]
    out_t = _final(mx2, z2, xt, g2, b2)
    return out_t.T[None]


# speedup vs baseline: 2.2111x; 1.0309x over previous
import jax
import jax.numpy as jnp
from jax import lax
from jax.experimental import pallas as pl
from jax.experimental.pallas import tpu as pltpu
from jax.experimental.pallas import tpu_sc as plsc

N = 10000
D = 64
K = 32
NC, NS = 2, 16
NPAD = 10240
CN = 32
NB0 = 320
NB1 = 320
NCH0 = NB0 // CN
NCH1 = NB1 // CN
ROWS = CN * K
CORE0_NODES = NS * NB0

_SQRT2 = 1.4142135623730951
_F32 = jnp.float32
_BF16 = jnp.bfloat16


def _gelu(v):
    return v * 0.5 * (1.0 + lax.erf(v / _SQRT2))


def _bn(m, g, b):
    mu = jnp.mean(m, axis=0, keepdims=True)
    var = jnp.mean((m - mu) ** 2, axis=0, keepdims=True)
    return (m - mu) * lax.rsqrt(var + 1e-5) * g + b


_DIMS = (((1,), (1,)), ((), ()))


def _mm_body(xt_ref, w_ref, y_ref, z_ref):
    xt = xt_ref[...]
    w = w_ref[...]
    a = w[:, :D]
    bz = w[:, D:] - a
    y_ref[...] = lax.dot_general(
        xt, a, _DIMS, preferred_element_type=_F32).astype(_BF16)
    z_ref[...] = lax.dot_general(xt, bz, _DIMS, preferred_element_type=_F32)


_mm = pl.pallas_call(
    _mm_body,
    out_shape=(jax.ShapeDtypeStruct((N, D), _BF16),
               jax.ShapeDtypeStruct((N, D), _F32)),
)


def _bnmm_body(mx_ref, z_ref, w_ref, g_ref, b_ref, y2_ref, z2_ref):
    m = mx_ref[...].astype(_F32) + z_ref[...]
    h = _gelu(_bn(m, g_ref[...], b_ref[...]))
    w = w_ref[...]
    a = w[:, :D]
    bz = w[:, D:] - a
    y2_ref[...] = lax.dot_general(
        h, a, _DIMS, preferred_element_type=_F32).astype(_BF16)
    z2_ref[...] = lax.dot_general(h, bz, _DIMS, preferred_element_type=_F32)


_bnmm = pl.pallas_call(
    _bnmm_body,
    out_shape=(jax.ShapeDtypeStruct((N, D), _BF16),
               jax.ShapeDtypeStruct((N, D), _F32)),
)


def _final_body(mx_ref, z_ref, xt_ref, g_ref, b_ref, o_ref):
    s = xt_ref[...] + mx_ref[...].astype(_F32) + z_ref[...]
    o_ref[...] = _gelu(_bn(s, g_ref[...], b_ref[...]))


_final = pl.pallas_call(
    _final_body,
    out_shape=jax.ShapeDtypeStruct((N, D), _F32),
)


def _sc_gathermax_body(y_hbm, nbr_hbm, out_hbm, idx_v,
                       tbl, g0, g1, ob0, ob1, sg0, sg1, sh0, sh1,
                       so0, so1):
    cid = lax.axis_index("c")
    sid = lax.axis_index("s")
    node_base = jnp.where(cid == 0, sid * NB0, CORE0_NODES + sid * NB1)
    npairs = jnp.where(cid == 0, NCH0 // 2, NCH1 // 2)
    idx_base = node_base * K

    gs = (g0, g1)
    obs = (ob0, ob1)
    sgs = (sg0, sg1)
    shs = (sh0, sh1)
    sos = (so0, so1)
    HR = ROWS // 2

    @pl.when(cid == 0)
    def _copy_idx0():
        pltpu.sync_copy(nbr_hbm.at[pl.ds(idx_base, NB0 * K)], idx_v)

    @pl.when(cid == 1)
    def _copy_idx1():
        pltpu.sync_copy(nbr_hbm.at[pl.ds(idx_base, NB1 * K)],
                        idx_v.at[pl.ds(0, NB1 * K)])

    pltpu.sync_copy(y_hbm.at[pl.ds(sid * (N // NS), N // NS)],
                    tbl.at[pl.ds(sid * (N // NS), N // NS)])
    plsc.subcore_barrier()

    def issue(c, b):
        pltpu.async_copy(tbl.at[idx_v.at[pl.ds(c * ROWS, HR)]],
                         gs[b].at[pl.ds(0, HR)], sgs[b])
        pltpu.async_copy(tbl.at[idx_v.at[pl.ds(c * ROWS + HR, HR)]],
                         gs[b].at[pl.ds(HR, HR)], shs[b])

    issue(0, 0)
    issue(1, 1)

    def pair(c2, carry):
        for b in range(2):
            c = 2 * c2 + b
            gbuf = gs[b]
            obuf = obs[b]
            pltpu.make_async_copy(y_hbm.at[pl.ds(0, HR)],
                                  gbuf.at[pl.ds(0, HR)], sgs[b]).wait()
            pltpu.make_async_copy(y_hbm.at[pl.ds(0, HR)],
                                  gbuf.at[pl.ds(HR, HR)], shs[b]).wait()

            @pl.when(c2 > 0)
            def _drain_store():
                pltpu.make_async_copy(obuf, out_hbm.at[pl.ds(0, CN)],
                                      sos[b]).wait()

            def node(i, carry2):
                base = i * K
                for q in range(D // 32):
                    m = gbuf[base, pl.ds(q * 32, 32)]
                    for r in range(1, K):
                        m = jnp.maximum(m, gbuf[base + r, pl.ds(q * 32, 32)])
                    obuf[i, pl.ds(q * 32, 32)] = m
                return carry2

            lax.fori_loop(0, CN, node, 0)
            row0 = node_base + c * CN
            pltpu.async_copy(obuf, out_hbm.at[pl.ds(row0, CN)], sos[b])

            nxt = c + 2

            @pl.when(nxt < 2 * npairs)
            def _issue_next():
                issue(nxt, b)
        return carry

    lax.fori_loop(0, npairs, pair, 0)
    for b in range(2):
        pltpu.make_async_copy(obs[b], out_hbm.at[pl.ds(0, CN)],
                              sos[b]).wait()


_gathermax = pl.kernel(
    _sc_gathermax_body,
    out_type=jax.ShapeDtypeStruct((NPAD, D), _BF16),
    mesh=plsc.VectorSubcoreMesh(core_axis_name="c", subcore_axis_name="s",
                                num_cores=NC, num_subcores=NS),
    scratch_types=[
        pltpu.VMEM((NB0 * K,), jnp.int32),
        pltpu.VMEM_SHARED((N, D), _BF16),
        pltpu.VMEM((ROWS, D), _BF16),
        pltpu.VMEM((ROWS, D), _BF16),
        pltpu.VMEM((CN, D), _BF16),
        pltpu.VMEM((CN, D), _BF16),
        pltpu.SemaphoreType.DMA,
        pltpu.SemaphoreType.DMA,
        pltpu.SemaphoreType.DMA,
        pltpu.SemaphoreType.DMA,
        pltpu.SemaphoreType.DMA,
        pltpu.SemaphoreType.DMA,
    ],
    compiler_params=pltpu.CompilerParams(use_tc_tiling_on_sc=False),
)


def kernel(x, neighbor_ind, W1, W2, gamma1, beta1, gamma2, beta2):
    xt = x[0].T
    nbr = neighbor_ind[0].astype(jnp.int32)
    nbr_flat = jnp.pad(nbr, ((0, NPAD - N), (0, 0))).reshape(-1)
    g1 = gamma1.reshape(1, D)
    b1 = beta1.reshape(1, D)
    g2 = gamma2.reshape(1, D)
    b2 = beta2.reshape(1, D)

    y1, z1 = _mm(xt, W1)
    mx1 = _gathermax(y1, nbr_flat)[:N]
    y2, z2 = _bnmm(mx1, z1, W2, g1, b1)
    mx2 = _gathermax(y2, nbr_flat)[:N]
    out_t = _final(mx2, z2, xt, g2, b2)
    return out_t.T[None]


# speedup vs baseline: 2.4419x; 1.1044x over previous
import jax
import jax.numpy as jnp
from jax import lax
from jax.experimental import pallas as pl
from jax.experimental.pallas import tpu as pltpu
from jax.experimental.pallas import tpu_sc as plsc

N = 10000
D = 64
K = 32
NC, NS = 2, 16
NPAD = 10240
CN = 32
NB0 = 320
NB1 = 320
NCH0 = NB0 // CN
NCH1 = NB1 // CN
ROWS = CN * K
CORE0_NODES = NS * NB0

_SQRT2 = 1.4142135623730951
_F32 = jnp.float32
_BF16 = jnp.bfloat16


def _gelu(v):
    return v * 0.5 * (1.0 + lax.erf(v / _SQRT2))


def _bn(m, g, b):
    mu = jnp.mean(m, axis=0, keepdims=True)
    var = jnp.mean((m - mu) ** 2, axis=0, keepdims=True)
    return (m - mu) * lax.rsqrt(var + 1e-5) * g + b


_DIMS = (((1,), (1,)), ((), ()))
_DIMS_T = (((0,), (1,)), ((), ()))


def _mm_body(x_ref, w_ref, y_ref, z_ref):
    xc = x_ref[...]
    w = w_ref[...]
    a = w[:, :D]
    bz = w[:, D:] - a
    y_ref[...] = lax.dot_general(
        xc, a, _DIMS_T, preferred_element_type=_F32).astype(_BF16)
    z_ref[...] = lax.dot_general(xc, bz, _DIMS_T, preferred_element_type=_F32)


_mm = pl.pallas_call(
    _mm_body,
    out_shape=(jax.ShapeDtypeStruct((N, D), _BF16),
               jax.ShapeDtypeStruct((N, D), _F32)),
)


def _bnmm_body(mx_ref, z_ref, w_ref, g_ref, b_ref, y2_ref, z2_ref):
    m = mx_ref[pl.ds(0, N), :].astype(_F32) + z_ref[...]
    h = _gelu(_bn(m, g_ref[...], b_ref[...]))
    w = w_ref[...]
    a = w[:, :D]
    bz = w[:, D:] - a
    y2_ref[...] = lax.dot_general(
        h, a, _DIMS, preferred_element_type=_F32).astype(_BF16)
    z2_ref[...] = lax.dot_general(h, bz, _DIMS, preferred_element_type=_F32)


_bnmm = pl.pallas_call(
    _bnmm_body,
    out_shape=(jax.ShapeDtypeStruct((N, D), _BF16),
               jax.ShapeDtypeStruct((N, D), _F32)),
)


def _final_body(mx_ref, z_ref, x_ref, g_ref, b_ref, o_ref):
    t = mx_ref[pl.ds(0, N), :].astype(_F32) + z_ref[...]
    s = x_ref[...] + t.T
    mu = jnp.mean(s, axis=1, keepdims=True)
    var = jnp.mean((s - mu) ** 2, axis=1, keepdims=True)
    o_ref[...] = _gelu(
        (s - mu) * lax.rsqrt(var + 1e-5) * g_ref[...] + b_ref[...])


_final = pl.pallas_call(
    _final_body,
    out_shape=jax.ShapeDtypeStruct((D, N), _F32),
)


def _sc_gathermax_body(y_hbm, nbr_hbm, out_hbm, idx_v,
                       tbl, g0, g1, ob0, ob1, sg0, sg1, sh0, sh1,
                       so0, so1):
    cid = lax.axis_index("c")
    sid = lax.axis_index("s")
    node_base = jnp.where(cid == 0, sid * NB0, CORE0_NODES + sid * NB1)
    npairs = jnp.where(cid == 0, NCH0 // 2, NCH1 // 2)
    idx_base = node_base * K

    gs = (g0, g1)
    obs = (ob0, ob1)
    sgs = (sg0, sg1)
    shs = (sh0, sh1)
    sos = (so0, so1)
    HR = ROWS // 2

    @pl.when(cid == 0)
    def _copy_idx0():
        pltpu.sync_copy(nbr_hbm.at[pl.ds(idx_base, NB0 * K)], idx_v)

    @pl.when(cid == 1)
    def _copy_idx1():
        pltpu.sync_copy(nbr_hbm.at[pl.ds(idx_base, NB1 * K)],
                        idx_v.at[pl.ds(0, NB1 * K)])

    pltpu.sync_copy(y_hbm.at[pl.ds(sid * (N // NS), N // NS)],
                    tbl.at[pl.ds(sid * (N // NS), N // NS)])
    plsc.subcore_barrier()

    def issue(c, b):
        pltpu.async_copy(tbl.at[idx_v.at[pl.ds(c * ROWS, HR)]],
                         gs[b].at[pl.ds(0, HR)], sgs[b])
        pltpu.async_copy(tbl.at[idx_v.at[pl.ds(c * ROWS + HR, HR)]],
                         gs[b].at[pl.ds(HR, HR)], shs[b])

    issue(0, 0)
    issue(1, 1)

    def pair(c2, carry):
        for b in range(2):
            c = 2 * c2 + b
            gbuf = gs[b]
            obuf = obs[b]
            pltpu.make_async_copy(y_hbm.at[pl.ds(0, HR)],
                                  gbuf.at[pl.ds(0, HR)], sgs[b]).wait()
            pltpu.make_async_copy(y_hbm.at[pl.ds(0, HR)],
                                  gbuf.at[pl.ds(HR, HR)], shs[b]).wait()

            @pl.when(c2 > 0)
            def _drain_store():
                pltpu.make_async_copy(obuf, out_hbm.at[pl.ds(0, CN)],
                                      sos[b]).wait()

            def node(i, carry2):
                base = i * K
                for q in range(D // 32):
                    m = gbuf[base, pl.ds(q * 32, 32)]
                    for r in range(1, K):
                        m = jnp.maximum(m, gbuf[base + r, pl.ds(q * 32, 32)])
                    obuf[i, pl.ds(q * 32, 32)] = m
                return carry2

            lax.fori_loop(0, CN, node, 0)
            row0 = node_base + c * CN
            pltpu.async_copy(obuf, out_hbm.at[pl.ds(row0, CN)], sos[b])

            nxt = c + 2

            @pl.when(nxt < 2 * npairs)
            def _issue_next():
                issue(nxt, b)
        return carry

    lax.fori_loop(0, npairs, pair, 0)
    for b in range(2):
        pltpu.make_async_copy(obs[b], out_hbm.at[pl.ds(0, CN)],
                              sos[b]).wait()


_gathermax = pl.kernel(
    _sc_gathermax_body,
    out_type=jax.ShapeDtypeStruct((NPAD, D), _BF16),
    mesh=plsc.VectorSubcoreMesh(core_axis_name="c", subcore_axis_name="s",
                                num_cores=NC, num_subcores=NS),
    scratch_types=[
        pltpu.VMEM((NB0 * K,), jnp.int32),
        pltpu.VMEM_SHARED((N, D), _BF16),
        pltpu.VMEM((ROWS, D), _BF16),
        pltpu.VMEM((ROWS, D), _BF16),
        pltpu.VMEM((CN, D), _BF16),
        pltpu.VMEM((CN, D), _BF16),
        pltpu.SemaphoreType.DMA,
        pltpu.SemaphoreType.DMA,
        pltpu.SemaphoreType.DMA,
        pltpu.SemaphoreType.DMA,
        pltpu.SemaphoreType.DMA,
        pltpu.SemaphoreType.DMA,
    ],
    compiler_params=pltpu.CompilerParams(use_tc_tiling_on_sc=False),
)


def kernel(x, neighbor_ind, W1, W2, gamma1, beta1, gamma2, beta2):
    x2d = x[0]
    nbr = neighbor_ind[0].astype(jnp.int32)
    nbr_flat = jnp.pad(nbr, ((0, NPAD - N), (0, 0))).reshape(-1)
    g1 = gamma1.reshape(1, D)
    b1 = beta1.reshape(1, D)
    g2 = gamma2.reshape(D, 1)
    b2 = beta2.reshape(D, 1)

    y1, z1 = _mm(x2d, W1)
    mx1 = _gathermax(y1, nbr_flat)
    y2, z2 = _bnmm(mx1, z1, W2, g1, b1)
    mx2 = _gathermax(y2, nbr_flat)
    out2d = _final(mx2, z2, x2d, g2, b2)
    return out2d[None]
